# Initial kernel scaffold; baseline (speedup 1.0000x reference)
import jax
import jax.numpy as jnp
from jax.experimental import pallas as pl


def kernel(x, edge_index, edge_attr, W1, b1, W2, b2):
    raise NotImplementedError("write your pallas kernel here")


# speedup vs baseline: 10.7741x; 10.7741x over previous
import math

import jax
import jax.numpy as jnp
from jax import lax
from jax.experimental import pallas as pl
from jax.experimental.pallas import tpu as pltpu
from jax.experimental.pallas import tpu_sc as plsc

NC = 2
NS = 16
L = 16
NT = NC * NS
B = 128
BD = 128
NBUF = 4
NPH = 2


def _bcast_lane(vec, lane):
    dn = lax.GatherDimensionNumbers(
        offset_dims=(), collapsed_slice_dims=(0,), start_index_map=(0,)
    )
    idx = jnp.full((L, 1), lane, jnp.int32)
    return lax.gather(vec, idx, dn, (1,),
                      mode=lax.GatherScatterMode.PROMISE_IN_BOUNDS)


def _mesh():
    return plsc.VectorSubcoreMesh(
        core_axis_name="c", subcore_axis_name="s", num_cores=NC, num_subcores=NS
    )


def _sc_deg(dst3, ew3, npad):
    nb = dst3.shape[1]

    def body(dst_hbm, ew_hbm, out_hbm, dstv, ewv, zb, degsh):
        cid = lax.axis_index("c")
        sid = lax.axis_index("s")
        tid = cid * NS + sid
        npt = npad // NS

        @pl.loop(0, npt // L)
        def _(i):
            zb[pl.ds(i * L, L)] = jnp.zeros((L,), jnp.float32)

        pltpu.sync_copy(zb, degsh.at[pl.ds(sid * npt, npt)])
        plsc.subcore_barrier()

        pltpu.sync_copy(dst_hbm.at[tid], dstv)
        pltpu.sync_copy(ew_hbm.at[tid], ewv)

        @pl.loop(0, nb)
        def _(b):
            pltpu.sync_copy(ewv.at[b], degsh.at[dstv.at[b]], add=True)

        plsc.subcore_barrier()
        pltpu.sync_copy(
            degsh.at[pl.ds(sid * npt, npt)],
            out_hbm.at[cid, pl.ds(sid * npt, npt)],
        )

    return pl.kernel(
        body,
        out_type=jax.ShapeDtypeStruct((NC, npad), jnp.float32),
        mesh=_mesh(),
        scratch_types=[
            pltpu.VMEM((nb, BD), jnp.int32),
            pltpu.VMEM((nb, BD), jnp.float32),
            pltpu.VMEM((npad // NS,), jnp.float32),
            pltpu.VMEM_SHARED((npad,), jnp.float32),
        ],
    )(dst3, ew3)


def _sc_agg(hp2, src4, dst4, ew3, npad):
    _, _, d2 = hp2.shape
    nb = src4.shape[2]
    rpt = npad // NS

    def body(hp_hbm, src_hbm, dst_hbm, ew_hbm, out_hbm,
             srcv, dstv, ewf, buf0, buf1, buf2, buf3,
             gs0, gs1, gs2, gs3, ss0, ss1, ss2, ss3, acc):
        cid = lax.axis_index("c")
        sid = lax.axis_index("s")
        bufs = (buf0, buf1, buf2, buf3)
        gsems = (gs0, gs1, gs2, gs3)
        ssems = (ss0, ss1, ss2, ss3)
        myhp = hp_hbm.at[cid]

        @pl.loop(0, B)
        def _(i):
            for j in range(d2 // L):
                buf0[i, pl.ds(j * L, L)] = jnp.zeros((L,), jnp.float32)

        @pl.loop(0, rpt, step=B)
        def _(r):
            pltpu.sync_copy(buf0, acc.at[pl.ds(sid * rpt + r, B)])

        plsc.subcore_barrier()
        nt_outer = nb // NBUF

        for p in range(NPH):
            pltpu.sync_copy(src_hbm.at[sid, p], srcv)
            pltpu.sync_copy(dst_hbm.at[sid, p], dstv)
            pltpu.sync_copy(ew_hbm.at[sid, p], ewf)

            @pl.loop(0, nt_outer)
            def _(t):
                for k in range(NBUF):
                    b = t * NBUF + k

                    @pl.when(t > 0)
                    def _():
                        pltpu.make_async_copy(
                            bufs[k], acc.at[dstv.at[b]], ssems[k]
                        ).wait()

                    pltpu.async_copy(myhp.at[srcv.at[b]], bufs[k], gsems[k])

                for k in range(NBUF):
                    b = t * NBUF + k
                    pltpu.make_async_copy(
                        myhp.at[srcv.at[b]], bufs[k], gsems[k]
                    ).wait()

                    @pl.loop(0, B)
                    def _(i):
                        ew_g = ewf[pl.ds(b * B + (i // L) * L, L)]
                        s = _bcast_lane(ew_g, i % L)
                        for j in range(d2 // L):
                            sl = pl.ds(j * L, L)
                            bufs[k][i, sl] = bufs[k][i, sl] * s

                    pltpu.async_copy(bufs[k], acc.at[dstv.at[b]], ssems[k],
                                     add=True)

            for k in range(NBUF):
                b = (nt_outer - 1) * NBUF + k
                pltpu.make_async_copy(bufs[k], acc.at[dstv.at[b]],
                                      ssems[k]).wait()

        plsc.subcore_barrier()

        @pl.loop(0, rpt, step=B)
        def _(r):
            pltpu.sync_copy(acc.at[pl.ds(sid * rpt + r, B)],
                            out_hbm.at[cid, pl.ds(sid * rpt + r, B)])

    return pl.kernel(
        body,
        out_type=jax.ShapeDtypeStruct((NC, npad, d2), jnp.float32),
        mesh=_mesh(),
        compiler_params=pltpu.CompilerParams(use_tc_tiling_on_sc=False),
        scratch_types=[
            pltpu.VMEM((nb, B), jnp.int32),
            pltpu.VMEM((nb, B), jnp.int32),
            pltpu.VMEM((nb * B,), jnp.float32),
            pltpu.VMEM((B, d2), jnp.float32),
            pltpu.VMEM((B, d2), jnp.float32),
            pltpu.VMEM((B, d2), jnp.float32),
            pltpu.VMEM((B, d2), jnp.float32),
            pltpu.SemaphoreType.DMA,
            pltpu.SemaphoreType.DMA,
            pltpu.SemaphoreType.DMA,
            pltpu.SemaphoreType.DMA,
            pltpu.SemaphoreType.DMA,
            pltpu.SemaphoreType.DMA,
            pltpu.SemaphoreType.DMA,
            pltpu.SemaphoreType.DMA,
            pltpu.VMEM_SHARED((npad, d2), jnp.float32),
        ],
    )(hp2, src4, dst4, ew3)


_RB = 1000


def _dis_of(pd):
    deg = 1.0 + pd[:, 0:1] + pd[:, 1:2]
    return lax.rsqrt(deg)


def _tc_prep(x, w, pdt):
    n, d = x.shape
    d2 = d // NC

    def body(x_ref, w_ref, pd_ref, o_ref):
        dis = _dis_of(pd_ref[...])
        h = dis * jnp.dot(x_ref[...], w_ref[...],
                          preferred_element_type=jnp.float32)
        o_ref[0] = h[:, :d2]
        o_ref[1] = h[:, d2:]

    return pl.pallas_call(
        body,
        grid=(n // _RB,),
        in_specs=[
            pl.BlockSpec((_RB, d), lambda i: (i, 0)),
            pl.BlockSpec((d, d), lambda i: (0, 0)),
            pl.BlockSpec((_RB, NC), lambda i: (i, 0)),
        ],
        out_specs=pl.BlockSpec((NC, _RB, d2), lambda i: (0, i, 0)),
        out_shape=jax.ShapeDtypeStruct((NC, n, d2), jnp.float32),
    )(x, w, pdt)


def _tc_mid(sp, hp2, pdt, b1, w2):
    _, n, d2 = hp2.shape
    d = d2 * NC

    def body(s_ref, hp_ref, pd_ref, b_ref, w_ref, o_ref):
        dis = _dis_of(pd_ref[...])
        t = jnp.concatenate(
            [dis * (s_ref[c] + hp_ref[c]) for c in range(NC)], axis=1
        )
        y = jnp.maximum(t + b_ref[...], 0.0)
        h = dis * jnp.dot(y, w_ref[...], preferred_element_type=jnp.float32)
        o_ref[0] = h[:, :d2]
        o_ref[1] = h[:, d2:]

    return pl.pallas_call(
        body,
        grid=(n // _RB,),
        in_specs=[
            pl.BlockSpec((NC, _RB, d2), lambda i: (0, i, 0)),
            pl.BlockSpec((NC, _RB, d2), lambda i: (0, i, 0)),
            pl.BlockSpec((_RB, NC), lambda i: (i, 0)),
            pl.BlockSpec((1, d), lambda i: (0, 0)),
            pl.BlockSpec((d, d), lambda i: (0, 0)),
        ],
        out_specs=pl.BlockSpec((NC, _RB, d2), lambda i: (0, i, 0)),
        out_shape=jax.ShapeDtypeStruct((NC, n, d2), jnp.float32),
    )(sp, hp2, pdt, b1, w2)


def _tc_final(sp, hp2, pdt, b2):
    _, n, d2 = hp2.shape
    d = d2 * NC

    def body(s_ref, hp_ref, pd_ref, b_ref, o_ref):
        dis = _dis_of(pd_ref[...])
        t = jnp.concatenate(
            [dis * (s_ref[c] + hp_ref[c]) for c in range(NC)], axis=1
        )
        o_ref[...] = jnp.maximum(t + b_ref[...], 0.0)

    return pl.pallas_call(
        body,
        grid=(n // _RB,),
        in_specs=[
            pl.BlockSpec((NC, _RB, d2), lambda i: (0, i, 0)),
            pl.BlockSpec((NC, _RB, d2), lambda i: (0, i, 0)),
            pl.BlockSpec((_RB, NC), lambda i: (i, 0)),
            pl.BlockSpec((1, d), lambda i: (0, 0)),
        ],
        out_specs=pl.BlockSpec((_RB, d), lambda i: (i, 0)),
        out_shape=jax.ShapeDtypeStruct((n, d), jnp.float32),
    )(sp, hp2, pdt, b2)


def kernel(x, edge_index, edge_attr, W1, b1, W2, b2):
    x = x.astype(jnp.float32)
    ew = edge_attr.astype(jnp.float32)
    n, d = x.shape
    e = edge_index.shape[1]

    src = edge_index[0].astype(jnp.int32)
    dst = edge_index[1].astype(jnp.int32)

    unit = math.lcm(NT * BD, NS * NPH * NBUF * B)
    epad = -(-e // unit) * unit
    src_p = jnp.pad(src, (0, epad - e))
    dst_p = jnp.pad(dst, (0, epad - e))
    ew_p = jnp.pad(ew, (0, epad - e))
    dst3 = dst_p.reshape(NT, epad // (NT * BD), BD)
    ew3d = ew_p.reshape(NT, epad // (NT * BD), BD)
    nba = epad // (NS * NPH * B)
    src4 = src_p.reshape(NS, NPH, nba, B)
    dst4 = dst_p.reshape(NS, NPH, nba, B)
    ew3a = ew_p.reshape(NS, NPH, nba * B)

    npad = -(-n // (NS * BD)) * (NS * BD)
    pdeg = _sc_deg(dst3, ew3d, npad)
    pdt = pdeg.T[:n]

    h1p = _tc_prep(x, W1, pdt)
    s1 = _sc_agg(h1p, src4, dst4, ew3a, npad)
    h2p = _tc_mid(s1, h1p, pdt, b1.reshape(1, d), W2)
    s2 = _sc_agg(h2p, src4, dst4, ew3a, npad)
    return _tc_final(s2, h2p, pdt, b2.reshape(1, d))
